# baseline (speedup 1.0000x reference)
import jax
import jax.numpy as jnp
from jax.experimental import pallas as pl


def kernel(x, edge_index, edge_attr, batch, W_emb, b_emb, W_conv, b_conv,
           bn_g, bn_b, ln_g, ln_b, W_c2f, b_c2f, W_fc, b_fc, W_out, b_out):
    L = W_conv.shape[0]
    G = 256
    h = x @ W_emb.T + b_emb
    src = edge_index[0]
    dst = edge_index[1]
    for l in range(L):
        z = jnp.concatenate([h[dst], h[src], edge_attr], axis=1)
        z = z @ W_conv[l].T + b_conv[l]
        mu = z.mean(axis=0)
        var = z.var(axis=0)
        z = (z - mu) / jnp.sqrt(var + 1e-5) * bn_g[l] + bn_b[l]
        z1, z2 = jnp.split(z, 2, axis=1)
        m = jax.nn.sigmoid(z1) * jax.nn.softplus(z2)
        agg = jnp.zeros_like(h).at[dst].add(m)
        mu2 = agg.mean(axis=-1, keepdims=True)
        var2 = agg.var(axis=-1, keepdims=True)
        agg = (agg - mu2) / jnp.sqrt(var2 + 1e-5) * ln_g[l] + ln_b[l]
        h = jax.nn.softplus(agg + h)
    sums = jax.ops.segment_sum(h, batch, num_segments=G)
    cnt = jax.ops.segment_sum(jnp.ones((h.shape[0], 1), dtype=h.dtype), batch, num_segments=G)
    mol = sums / jnp.clip(cnt, 1.0, None)
    mol = jax.nn.softplus(mol @ W_c2f.T + b_c2f)
    for i in range(W_fc.shape[0]):
        mol = jax.nn.softplus(mol @ W_fc[i].T + b_fc[i])
    out = mol @ W_out.T + b_out
    return out


# speedup vs baseline: 1.2783x; 1.2783x over previous
import functools

import jax
import jax.numpy as jnp
from jax import lax
from jax.experimental import pallas as pl
from jax.experimental.pallas import tpu as pltpu
from jax.experimental.pallas import tpu_sc as plsc

N = 10000
E = 320000
H = 64
HH = 128
DE = 16
G = 256
NC = 2
NS = 16
NW = NC * NS
EW = E // NW
K = 80
NCH = EW // K
NPT = N // NS

_mesh = plsc.VectorSubcoreMesh(
    core_axis_name="c", subcore_axis_name="s", num_cores=NC, num_subcores=NS)


def _edge_z_body(p_hbm, q_hbm, ep_hbm, dst_hbm, src_hbm,
                 z_hbm, stats_hbm,
                 idx_d, idx_s, pb, qb, eb, statsb, sem1, sem2, sem3):
    c = lax.axis_index("c")
    s = lax.axis_index("s")
    wid = c * NS + s
    base0 = wid * EW
    pltpu.sync_copy(dst_hbm.at[wid], idx_d)
    pltpu.sync_copy(src_hbm.at[wid], idx_s)

    def chunk_body(j, acc):
        base = base0 + j * K
        cp1 = pltpu.async_copy(p_hbm.at[idx_d.at[j]], pb, sem1)
        cp2 = pltpu.async_copy(q_hbm.at[idx_s.at[j]], qb, sem2)
        cp3 = pltpu.async_copy(ep_hbm.at[pl.ds(base, K), :], eb, sem3)
        cp1.wait()
        cp2.wait()
        cp3.wait()

        def row_body(r, a):
            out = []
            for g in range(8):
                sl = pl.ds(g * 16, 16)
                zv = pb[r, sl] + qb[r, sl] + eb[r, sl]
                eb[r, sl] = zv
                out.append((a[g] + zv, a[8 + g] + zv * zv))
            return tuple(x for x, _ in out) + tuple(y for _, y in out)

        acc = lax.fori_loop(0, K, row_body, acc)
        pltpu.sync_copy(eb, z_hbm.at[pl.ds(base, K), :])
        return acc

    acc0 = tuple(jnp.zeros((16,), jnp.float32) for _ in range(16))
    acc = lax.fori_loop(0, NCH, chunk_body, acc0)
    for g in range(8):
        sl = pl.ds(g * 16, 16)
        statsb[0, sl] = acc[g]
        statsb[1, sl] = acc[8 + g]
    pltpu.sync_copy(statsb, stats_hbm.at[wid])


_edge_z = functools.partial(
    pl.kernel, _edge_z_body, mesh=_mesh,
    out_type=[jax.ShapeDtypeStruct((E, HH), jnp.float32),
              jax.ShapeDtypeStruct((NW, 2, HH), jnp.float32)],
    scratch_types=[pltpu.VMEM((NCH, K), jnp.int32),
                   pltpu.VMEM((NCH, K), jnp.int32),
                   pltpu.VMEM((K, HH), jnp.float32),
                   pltpu.VMEM((K, HH), jnp.float32),
                   pltpu.VMEM((K, HH), jnp.float32),
                   pltpu.VMEM((2, HH), jnp.float32),
                   pltpu.SemaphoreType.DMA,
                   pltpu.SemaphoreType.DMA,
                   pltpu.SemaphoreType.DMA])()


NFG = 8
FW = H // NFG
NEG = NW // NFG
EE = E // NEG
CK = 2000
SCH = EE // CK


def _scatter_body(m_hbm, dst_hbm, agg_hbm, ib, mb, acc):
    c = lax.axis_index("c")
    s = lax.axis_index("s")
    wid = c * NS + s
    eg = wid // NFG
    fg = wid % NFG

    def zrow(i, _):
        acc[pl.ds(i * 16, 16)] = jnp.zeros((16,), jnp.float32)
        return 0
    lax.fori_loop(0, N * FW // 16, zrow, 0)

    lane8 = lax.iota(jnp.int32, 16) * FW

    def chunk_body(ch, _):
        pltpu.sync_copy(dst_hbm.at[eg, ch], ib)
        pltpu.sync_copy(m_hbm.at[eg * SCH + ch, fg], mb)

        def grp(g, _):
            d16 = ib[pl.ds(g * 16, 16)]
            a0 = d16 * FW
            base = g * (16 * FW)
            for f in range(FW):
                vals = plsc.load_gather(mb, [lane8 + (base + f)])
                plsc.addupdate_scatter(acc, [a0 + f], vals)
            return 0

        lax.fori_loop(0, CK // 16, grp, 0)
        return 0

    lax.fori_loop(0, SCH, chunk_body, 0)
    pltpu.sync_copy(acc, agg_hbm.at[eg, fg])


_scatter = functools.partial(
    pl.kernel, _scatter_body, mesh=_mesh,
    out_type=[jax.ShapeDtypeStruct((NEG, NFG, N * FW), jnp.float32)],
    compiler_params=pltpu.CompilerParams(needs_layout_passes=False),
    scratch_types=[pltpu.VMEM((CK,), jnp.int32),
                   pltpu.VMEM((CK * FW,), jnp.float32),
                   pltpu.VMEM((N * FW,), jnp.float32)])()


def _softplus(v):
    return jnp.maximum(v, 0.0) + jnp.log1p(jnp.exp(-jnp.abs(v)))


def _embed_body(x_ref, w_ref, b_ref, o_ref):
    o_ref[...] = lax.dot_general(
        x_ref[...], w_ref[...], (((1,), (1,)), ((), ())),
        preferred_element_type=jnp.float32) + b_ref[...]


def _embed(x, w, b):
    blk = 2000
    return pl.pallas_call(
        _embed_body,
        grid=(N // blk,),
        in_specs=[pl.BlockSpec((blk, 128), lambda i: (i, 0)),
                  pl.BlockSpec((H, 128), lambda i: (0, 0)),
                  pl.BlockSpec((1, H), lambda i: (0, 0))],
        out_specs=pl.BlockSpec((blk, H), lambda i: (i, 0)),
        out_shape=jax.ShapeDtypeStruct((N, H), jnp.float32),
    )(x, w, b)


def _pq_body(h_ref, wd_ref, ws_ref, p_ref, q_ref):
    h = h_ref[...]
    p_ref[...] = lax.dot_general(h, wd_ref[...], (((1,), (1,)), ((), ())),
                                 preferred_element_type=jnp.float32)
    q_ref[...] = lax.dot_general(h, ws_ref[...], (((1,), (1,)), ((), ())),
                                 preferred_element_type=jnp.float32)


def _pq(h, wd, ws):
    blk = 2000
    return pl.pallas_call(
        _pq_body,
        grid=(N // blk,),
        in_specs=[pl.BlockSpec((blk, H), lambda i: (i, 0)),
                  pl.BlockSpec((HH, H), lambda i: (0, 0)),
                  pl.BlockSpec((HH, H), lambda i: (0, 0))],
        out_specs=[pl.BlockSpec((blk, HH), lambda i: (i, 0)),
                   pl.BlockSpec((blk, HH), lambda i: (i, 0))],
        out_shape=[jax.ShapeDtypeStruct((N, HH), jnp.float32),
                   jax.ShapeDtypeStruct((N, HH), jnp.float32)],
    )(h, wd, ws)


def _ep_body(e_ref, w_ref, b_ref, o_ref):
    o_ref[...] = lax.dot_general(
        e_ref[...], w_ref[...], (((1,), (1,)), ((), ())),
        preferred_element_type=jnp.float32) + b_ref[...]


def _ep(ea, we, b):
    blk = 4000
    return pl.pallas_call(
        _ep_body,
        grid=(E // blk,),
        in_specs=[pl.BlockSpec((blk, DE), lambda i: (i, 0)),
                  pl.BlockSpec((HH, DE), lambda i: (0, 0)),
                  pl.BlockSpec((1, HH), lambda i: (0, 0))],
        out_specs=pl.BlockSpec((blk, HH), lambda i: (i, 0)),
        out_shape=jax.ShapeDtypeStruct((E, HH), jnp.float32),
    )(ea, we, b)


def _gate_body(z_ref, st_ref, g_ref, b_ref, m_ref):
    st = st_ref[...]
    mu = jnp.sum(st[:, 0, :], axis=0, keepdims=True) * (1.0 / E)
    msq = jnp.sum(st[:, 1, :], axis=0, keepdims=True) * (1.0 / E)
    var = msq - mu * mu
    inv = lax.rsqrt(var + 1e-5)
    zn = (z_ref[...] - mu) * inv * g_ref[...] + b_ref[...]
    z1 = zn[:, :H]
    z2 = zn[:, H:]
    m = (1.0 / (1.0 + jnp.exp(-z1))) * _softplus(z2)
    blk = m.shape[0]
    m_ref[...] = jnp.transpose(m.reshape(blk, NFG, FW),
                               (1, 0, 2)).reshape(1, NFG, blk * FW)


def _gate(z, stats, g, b):
    blk = CK
    return pl.pallas_call(
        _gate_body,
        grid=(E // blk,),
        in_specs=[pl.BlockSpec((blk, HH), lambda i: (i, 0)),
                  pl.BlockSpec((NW, 2, HH), lambda i: (0, 0, 0)),
                  pl.BlockSpec((1, HH), lambda i: (0, 0)),
                  pl.BlockSpec((1, HH), lambda i: (0, 0))],
        out_specs=pl.BlockSpec((1, NFG, blk * FW), lambda i: (i, 0, 0)),
        out_shape=jax.ShapeDtypeStruct((E // blk, NFG, blk * FW),
                                       jnp.float32),
    )(z, stats, g, b)


def _node_body(agg_ref, h_ref, g_ref, b_ref, o_ref):
    ap = agg_ref[...]
    asum = ap[0] + ap[1] + ap[2] + ap[3]
    blk = h_ref.shape[0]
    agg = jnp.transpose(asum.reshape(NFG, blk, FW),
                        (1, 0, 2)).reshape(blk, H)
    mu = jnp.mean(agg, axis=-1, keepdims=True)
    d = agg - mu
    var = jnp.mean(d * d, axis=-1, keepdims=True)
    an = d * lax.rsqrt(var + 1e-5) * g_ref[...] + b_ref[...]
    o_ref[...] = _softplus(an + h_ref[...])


def _node_update(aggp, h, g, b):
    blk = 2000
    return pl.pallas_call(
        _node_body,
        grid=(N // blk,),
        in_specs=[pl.BlockSpec((NEG, NFG, blk * FW), lambda i: (0, 0, i)),
                  pl.BlockSpec((blk, H), lambda i: (i, 0)),
                  pl.BlockSpec((1, H), lambda i: (0, 0)),
                  pl.BlockSpec((1, H), lambda i: (0, 0))],
        out_specs=pl.BlockSpec((blk, H), lambda i: (i, 0)),
        out_shape=jax.ShapeDtypeStruct((N, H), jnp.float32),
    )(aggp, h, g, b)


def _pool_body(h_ref, b_ref, s_ref, c_ref):
    i = pl.program_id(0)

    @pl.when(i == 0)
    def _():
        s_ref[...] = jnp.zeros_like(s_ref)
        c_ref[...] = jnp.zeros_like(c_ref)

    blk = h_ref.shape[0]
    ids = b_ref[...]
    onehot = (ids == lax.broadcasted_iota(jnp.int32, (blk, G), 1)
              ).astype(jnp.float32)
    s_ref[...] += lax.dot_general(onehot, h_ref[...],
                                  (((0,), (0,)), ((), ())),
                                  preferred_element_type=jnp.float32)
    c_ref[...] += lax.dot_general(onehot, jnp.ones((blk, 1), jnp.float32),
                                  (((0,), (0,)), ((), ())),
                                  preferred_element_type=jnp.float32)


def _pool(h, batch2):
    blk = 2000
    return pl.pallas_call(
        _pool_body,
        grid=(N // blk,),
        in_specs=[pl.BlockSpec((blk, H), lambda i: (i, 0)),
                  pl.BlockSpec((blk, 1), lambda i: (i, 0))],
        out_specs=[pl.BlockSpec((G, H), lambda i: (0, 0)),
                   pl.BlockSpec((G, 1), lambda i: (0, 0))],
        out_shape=[jax.ShapeDtypeStruct((G, H), jnp.float32),
                   jax.ShapeDtypeStruct((G, 1), jnp.float32)],
    )(h, batch2)


def _head_body(s_ref, c_ref, wc_ref, bc_ref, wf_ref, bf_ref, wo_ref, bo_ref,
               o_ref):
    mol = s_ref[...] / jnp.maximum(c_ref[...], 1.0)
    mol = _softplus(lax.dot_general(mol, wc_ref[...], (((1,), (1,)), ((), ())),
                                    preferred_element_type=jnp.float32)
                    + bc_ref[...])
    for i in range(wf_ref.shape[0]):
        mol = _softplus(lax.dot_general(mol, wf_ref[i],
                                        (((1,), (1,)), ((), ())),
                                        preferred_element_type=jnp.float32)
                        + bf_ref[i][None, :])
    o_ref[...] = (jnp.sum(mol * wo_ref[...], axis=1, keepdims=True)
                  + bo_ref[0, 0])


def _head(sums, cnt, wc, bc, wf, bf, wo, bo):
    nfc = wf.shape[0]
    fc = wf.shape[1]
    return pl.pallas_call(
        _head_body,
        grid=(1,),
        in_specs=[pl.BlockSpec((G, H), lambda i: (0, 0)),
                  pl.BlockSpec((G, 1), lambda i: (0, 0)),
                  pl.BlockSpec((fc, H), lambda i: (0, 0)),
                  pl.BlockSpec((1, fc), lambda i: (0, 0)),
                  pl.BlockSpec((nfc, fc, fc), lambda i: (0, 0, 0)),
                  pl.BlockSpec((nfc, fc), lambda i: (0, 0)),
                  pl.BlockSpec((1, fc), lambda i: (0, 0)),
                  pl.BlockSpec((1, 1), lambda i: (0, 0))],
        out_specs=pl.BlockSpec((G, 1), lambda i: (0, 0)),
        out_shape=jax.ShapeDtypeStruct((G, 1), jnp.float32),
    )(sums, cnt, wc, bc, wf, bf, wo, bo)


def kernel(x, edge_index, edge_attr, batch, W_emb, b_emb, W_conv, b_conv,
           bn_g, bn_b, ln_g, ln_b, W_c2f, b_c2f, W_fc, b_fc, W_out, b_out):
    L = W_conv.shape[0]
    src2 = edge_index[0].reshape(NW, NCH, K).astype(jnp.int32)
    dst2 = edge_index[1].reshape(NW, NCH, K).astype(jnp.int32)
    dst5 = edge_index[1].reshape(NEG, SCH, CK).astype(jnp.int32)
    batch2 = batch.reshape(N, 1).astype(jnp.int32)

    h = _embed(x, W_emb, b_emb.reshape(1, H))
    for l in range(L):
        wd = W_conv[l, :, :H]
        ws = W_conv[l, :, H:2 * H]
        we = W_conv[l, :, 2 * H:]
        p, q = _pq(h, wd, ws)
        ep = _ep(edge_attr, we, b_conv[l].reshape(1, HH))
        z, stats = _edge_z(p, q, ep, dst2, src2)
        m = _gate(z, stats, bn_g[l].reshape(1, HH), bn_b[l].reshape(1, HH))
        (aggp,) = _scatter(m, dst5)
        h = _node_update(aggp, h, ln_g[l].reshape(1, H), ln_b[l].reshape(1, H))

    sums, cnt = _pool(h, batch2)
    out = _head(sums, cnt, W_c2f, b_c2f.reshape(1, -1), W_fc, b_fc,
                W_out, b_out.reshape(1, 1))
    return out


# speedup vs baseline: 1.4965x; 1.1707x over previous
import functools

import jax
import jax.numpy as jnp
from jax import lax
from jax.experimental import pallas as pl
from jax.experimental.pallas import tpu as pltpu
from jax.experimental.pallas import tpu_sc as plsc

N = 10000
E = 320000
H = 64
HH = 128
DE = 16
G = 256
NC = 2
NS = 16
NW = NC * NS
EW = E // NW
K = 80
NCH = EW // K
NPT = N // NS

_mesh = plsc.VectorSubcoreMesh(
    core_axis_name="c", subcore_axis_name="s", num_cores=NC, num_subcores=NS)


def _edge_z_body(p_hbm, q_hbm, ep_hbm, dst_hbm, src_hbm,
                 z_hbm, stats_hbm,
                 idx_d, idx_s, pb, qb, eb, zb, statsb, semp, semq, seme,
                 semz):
    c = lax.axis_index("c")
    s = lax.axis_index("s")
    wid = c * NS + s
    base0 = wid * EW
    pltpu.sync_copy(dst_hbm.at[wid], idx_d)
    pltpu.sync_copy(src_hbm.at[wid], idx_s)

    def issue(j, t):
        base = base0 + j * K
        pltpu.async_copy(p_hbm.at[idx_d.at[j]], pb[t], semp[t])
        pltpu.async_copy(q_hbm.at[idx_s.at[j]], qb[t], semq[t])
        pltpu.async_copy(ep_hbm.at[pl.ds(base, K), :], eb[t], seme[t])

    def wait_in(t):
        pltpu.make_async_copy(p_hbm.at[idx_d.at[0]], pb[t], semp[t]).wait()
        pltpu.make_async_copy(q_hbm.at[idx_s.at[0]], qb[t], semq[t]).wait()
        pltpu.make_async_copy(ep_hbm.at[pl.ds(0, K), :], eb[t],
                              seme[t]).wait()

    def compute(j, t, acc, first):
        wait_in(t)
        if not first:
            pltpu.make_async_copy(zb[t], z_hbm.at[pl.ds(0, K), :],
                                  semz[t]).wait()

        def row_body(r, a):
            out = []
            for g in range(8):
                sl = pl.ds(g * 16, 16)
                zv = pb[t][r, sl] + qb[t][r, sl] + eb[t][r, sl]
                zb[t][r, sl] = zv
                out.append((a[g] + zv, a[8 + g] + zv * zv))
            return tuple(x for x, _ in out) + tuple(y for _, y in out)

        acc = lax.fori_loop(0, K, row_body, acc)
        pltpu.async_copy(zb[t], z_hbm.at[pl.ds(base0 + j * K, K), :],
                         semz[t])
        return acc

    issue(0, 0)
    issue(1, 1)
    acc = tuple(jnp.zeros((16,), jnp.float32) for _ in range(16))

    def pair_body(jj, acc):
        j = 2 * jj
        acc = compute(j, 0, acc, first=False)
        issue(j + 2, 0)
        acc = compute(j + 1, 1, acc, first=False)

        @pl.when(jj < NCH // 2 - 1)
        def _():
            issue(j + 3, 1)

        return acc

    acc = compute(0, 0, acc, first=True)
    issue(2, 0)
    acc = compute(1, 1, acc, first=True)
    issue(3, 1)
    acc = lax.fori_loop(1, NCH // 2, pair_body, acc)
    acc = compute(NCH - 1, 0, acc, first=False)
    pltpu.make_async_copy(zb[0], z_hbm.at[pl.ds(0, K), :], semz[0]).wait()
    pltpu.make_async_copy(zb[1], z_hbm.at[pl.ds(0, K), :], semz[1]).wait()

    for g in range(8):
        sl = pl.ds(g * 16, 16)
        statsb[0, sl] = acc[g]
        statsb[1, sl] = acc[8 + g]
    pltpu.sync_copy(statsb, stats_hbm.at[wid])


_edge_z = functools.partial(
    pl.kernel, _edge_z_body, mesh=_mesh,
    out_type=[jax.ShapeDtypeStruct((E, HH), jnp.float32),
              jax.ShapeDtypeStruct((NW, 2, HH), jnp.float32)],
    scratch_types=[pltpu.VMEM((NCH, K), jnp.int32),
                   pltpu.VMEM((NCH, K), jnp.int32),
                   [pltpu.VMEM((K, HH), jnp.float32)] * 2,
                   [pltpu.VMEM((K, HH), jnp.float32)] * 2,
                   [pltpu.VMEM((K, HH), jnp.float32)] * 2,
                   [pltpu.VMEM((K, HH), jnp.float32)] * 2,
                   pltpu.VMEM((2, HH), jnp.float32),
                   [pltpu.SemaphoreType.DMA] * 2,
                   [pltpu.SemaphoreType.DMA] * 2,
                   [pltpu.SemaphoreType.DMA] * 2,
                   [pltpu.SemaphoreType.DMA] * 2])()


NFG = 8
FW = H // NFG
NEG = NW // NFG
EE = E // NEG
CK = 2000
SCH = EE // CK


GUNR = 5


def _scatter_body(m_hbm, dst_hbm, agg_hbm, ib, mb, acc, semi, semm):
    c = lax.axis_index("c")
    s = lax.axis_index("s")
    wid = c * NS + s
    eg = wid // NFG
    fg = wid % NFG

    def zrow(i, _):
        acc[pl.ds(i * 16, 16)] = jnp.zeros((16,), jnp.float32)
        return 0
    lax.fori_loop(0, N * FW // 16, zrow, 0)

    lane8 = lax.iota(jnp.int32, 16) * FW

    def issue(ch, t):
        pltpu.async_copy(dst_hbm.at[eg, ch], ib[t], semi[t])
        pltpu.async_copy(m_hbm.at[eg * SCH + ch, fg], mb[t], semm[t])

    def compute(t):
        pltpu.make_async_copy(dst_hbm.at[eg, 0], ib[t], semi[t]).wait()
        pltpu.make_async_copy(m_hbm.at[0, 0], mb[t], semm[t]).wait()

        def grp(gg, _):
            for u in range(GUNR):
                g = gg * GUNR + u
                d16 = ib[t][pl.ds(g * 16, 16)]
                a0 = d16 * FW
                base = g * (16 * FW)
                for f in range(FW):
                    vals = plsc.load_gather(mb[t], [lane8 + (base + f)])
                    plsc.addupdate_scatter(acc, [a0 + f], vals)
            return 0

        lax.fori_loop(0, CK // 16 // GUNR, grp, 0)

    issue(0, 0)
    issue(1, 1)

    def pair_body(jj, _):
        j = 2 * jj
        compute(0)

        @pl.when(jj < SCH // 2 - 1)
        def _():
            issue(j + 2, 0)

        compute(1)

        @pl.when(jj < SCH // 2 - 1)
        def _():
            issue(j + 3, 1)

        return 0

    lax.fori_loop(0, SCH // 2, pair_body, 0)
    pltpu.sync_copy(acc, agg_hbm.at[eg, fg])


_scatter = functools.partial(
    pl.kernel, _scatter_body, mesh=_mesh,
    out_type=[jax.ShapeDtypeStruct((NEG, NFG, N * FW), jnp.float32)],
    compiler_params=pltpu.CompilerParams(needs_layout_passes=False),
    scratch_types=[[pltpu.VMEM((CK,), jnp.int32)] * 2,
                   [pltpu.VMEM((CK * FW,), jnp.float32)] * 2,
                   pltpu.VMEM((N * FW,), jnp.float32),
                   [pltpu.SemaphoreType.DMA] * 2,
                   [pltpu.SemaphoreType.DMA] * 2])()


def _softplus(v):
    return jnp.maximum(v, 0.0) + jnp.log1p(jnp.exp(-jnp.abs(v)))


def _embed_body(x_ref, w_ref, b_ref, o_ref):
    o_ref[...] = lax.dot_general(
        x_ref[...], w_ref[...], (((1,), (1,)), ((), ())),
        preferred_element_type=jnp.float32) + b_ref[...]


def _embed(x, w, b):
    blk = 2000
    return pl.pallas_call(
        _embed_body,
        grid=(N // blk,),
        in_specs=[pl.BlockSpec((blk, 128), lambda i: (i, 0)),
                  pl.BlockSpec((H, 128), lambda i: (0, 0)),
                  pl.BlockSpec((1, H), lambda i: (0, 0))],
        out_specs=pl.BlockSpec((blk, H), lambda i: (i, 0)),
        out_shape=jax.ShapeDtypeStruct((N, H), jnp.float32),
    )(x, w, b)


def _pq_body(h_ref, wd_ref, ws_ref, p_ref, q_ref):
    h = h_ref[...]
    p_ref[...] = lax.dot_general(h, wd_ref[...], (((1,), (1,)), ((), ())),
                                 preferred_element_type=jnp.float32)
    q_ref[...] = lax.dot_general(h, ws_ref[...], (((1,), (1,)), ((), ())),
                                 preferred_element_type=jnp.float32)


def _pq(h, wd, ws):
    blk = 2000
    return pl.pallas_call(
        _pq_body,
        grid=(N // blk,),
        in_specs=[pl.BlockSpec((blk, H), lambda i: (i, 0)),
                  pl.BlockSpec((HH, H), lambda i: (0, 0)),
                  pl.BlockSpec((HH, H), lambda i: (0, 0))],
        out_specs=[pl.BlockSpec((blk, HH), lambda i: (i, 0)),
                   pl.BlockSpec((blk, HH), lambda i: (i, 0))],
        out_shape=[jax.ShapeDtypeStruct((N, HH), jnp.float32),
                   jax.ShapeDtypeStruct((N, HH), jnp.float32)],
    )(h, wd, ws)


def _ep_body(e_ref, w_ref, b_ref, o_ref):
    o_ref[...] = lax.dot_general(
        e_ref[...], w_ref[...], (((1,), (1,)), ((), ())),
        preferred_element_type=jnp.float32) + b_ref[...]


def _ep(ea, we, b):
    blk = 4000
    return pl.pallas_call(
        _ep_body,
        grid=(E // blk,),
        in_specs=[pl.BlockSpec((blk, DE), lambda i: (i, 0)),
                  pl.BlockSpec((HH, DE), lambda i: (0, 0)),
                  pl.BlockSpec((1, HH), lambda i: (0, 0))],
        out_specs=pl.BlockSpec((blk, HH), lambda i: (i, 0)),
        out_shape=jax.ShapeDtypeStruct((E, HH), jnp.float32),
    )(ea, we, b)


def _gate_body(z_ref, st_ref, g_ref, b_ref, m_ref):
    st = st_ref[...]
    mu = jnp.sum(st[:, 0, :], axis=0, keepdims=True) * (1.0 / E)
    msq = jnp.sum(st[:, 1, :], axis=0, keepdims=True) * (1.0 / E)
    var = msq - mu * mu
    inv = lax.rsqrt(var + 1e-5)
    zn = (z_ref[...] - mu) * inv * g_ref[...] + b_ref[...]
    z1 = zn[:, :H]
    z2 = zn[:, H:]
    m = (1.0 / (1.0 + jnp.exp(-z1))) * _softplus(z2)
    blk = m.shape[0]
    m_ref[...] = jnp.transpose(m.reshape(blk, NFG, FW),
                               (1, 0, 2)).reshape(1, NFG, blk * FW)


def _gate(z, stats, g, b):
    blk = CK
    return pl.pallas_call(
        _gate_body,
        grid=(E // blk,),
        in_specs=[pl.BlockSpec((blk, HH), lambda i: (i, 0)),
                  pl.BlockSpec((NW, 2, HH), lambda i: (0, 0, 0)),
                  pl.BlockSpec((1, HH), lambda i: (0, 0)),
                  pl.BlockSpec((1, HH), lambda i: (0, 0))],
        out_specs=pl.BlockSpec((1, NFG, blk * FW), lambda i: (i, 0, 0)),
        out_shape=jax.ShapeDtypeStruct((E // blk, NFG, blk * FW),
                                       jnp.float32),
    )(z, stats, g, b)


def _node_body(agg_ref, h_ref, g_ref, b_ref, o_ref):
    ap = agg_ref[...]
    asum = ap[0] + ap[1] + ap[2] + ap[3]
    blk = h_ref.shape[0]
    agg = jnp.transpose(asum.reshape(NFG, blk, FW),
                        (1, 0, 2)).reshape(blk, H)
    mu = jnp.mean(agg, axis=-1, keepdims=True)
    d = agg - mu
    var = jnp.mean(d * d, axis=-1, keepdims=True)
    an = d * lax.rsqrt(var + 1e-5) * g_ref[...] + b_ref[...]
    o_ref[...] = _softplus(an + h_ref[...])


def _node_update(aggp, h, g, b):
    blk = 2000
    return pl.pallas_call(
        _node_body,
        grid=(N // blk,),
        in_specs=[pl.BlockSpec((NEG, NFG, blk * FW), lambda i: (0, 0, i)),
                  pl.BlockSpec((blk, H), lambda i: (i, 0)),
                  pl.BlockSpec((1, H), lambda i: (0, 0)),
                  pl.BlockSpec((1, H), lambda i: (0, 0))],
        out_specs=pl.BlockSpec((blk, H), lambda i: (i, 0)),
        out_shape=jax.ShapeDtypeStruct((N, H), jnp.float32),
    )(aggp, h, g, b)


def _pool_body(h_ref, b_ref, s_ref, c_ref):
    i = pl.program_id(0)

    @pl.when(i == 0)
    def _():
        s_ref[...] = jnp.zeros_like(s_ref)
        c_ref[...] = jnp.zeros_like(c_ref)

    blk = h_ref.shape[0]
    ids = b_ref[...]
    onehot = (ids == lax.broadcasted_iota(jnp.int32, (blk, G), 1)
              ).astype(jnp.float32)
    s_ref[...] += lax.dot_general(onehot, h_ref[...],
                                  (((0,), (0,)), ((), ())),
                                  preferred_element_type=jnp.float32)
    c_ref[...] += lax.dot_general(onehot, jnp.ones((blk, 1), jnp.float32),
                                  (((0,), (0,)), ((), ())),
                                  preferred_element_type=jnp.float32)


def _pool(h, batch2):
    blk = 2000
    return pl.pallas_call(
        _pool_body,
        grid=(N // blk,),
        in_specs=[pl.BlockSpec((blk, H), lambda i: (i, 0)),
                  pl.BlockSpec((blk, 1), lambda i: (i, 0))],
        out_specs=[pl.BlockSpec((G, H), lambda i: (0, 0)),
                   pl.BlockSpec((G, 1), lambda i: (0, 0))],
        out_shape=[jax.ShapeDtypeStruct((G, H), jnp.float32),
                   jax.ShapeDtypeStruct((G, 1), jnp.float32)],
    )(h, batch2)


def _head_body(s_ref, c_ref, wc_ref, bc_ref, wf_ref, bf_ref, wo_ref, bo_ref,
               o_ref):
    mol = s_ref[...] / jnp.maximum(c_ref[...], 1.0)
    mol = _softplus(lax.dot_general(mol, wc_ref[...], (((1,), (1,)), ((), ())),
                                    preferred_element_type=jnp.float32)
                    + bc_ref[...])
    for i in range(wf_ref.shape[0]):
        mol = _softplus(lax.dot_general(mol, wf_ref[i],
                                        (((1,), (1,)), ((), ())),
                                        preferred_element_type=jnp.float32)
                        + bf_ref[i][None, :])
    o_ref[...] = (jnp.sum(mol * wo_ref[...], axis=1, keepdims=True)
                  + bo_ref[0, 0])


def _head(sums, cnt, wc, bc, wf, bf, wo, bo):
    nfc = wf.shape[0]
    fc = wf.shape[1]
    return pl.pallas_call(
        _head_body,
        grid=(1,),
        in_specs=[pl.BlockSpec((G, H), lambda i: (0, 0)),
                  pl.BlockSpec((G, 1), lambda i: (0, 0)),
                  pl.BlockSpec((fc, H), lambda i: (0, 0)),
                  pl.BlockSpec((1, fc), lambda i: (0, 0)),
                  pl.BlockSpec((nfc, fc, fc), lambda i: (0, 0, 0)),
                  pl.BlockSpec((nfc, fc), lambda i: (0, 0)),
                  pl.BlockSpec((1, fc), lambda i: (0, 0)),
                  pl.BlockSpec((1, 1), lambda i: (0, 0))],
        out_specs=pl.BlockSpec((G, 1), lambda i: (0, 0)),
        out_shape=jax.ShapeDtypeStruct((G, 1), jnp.float32),
    )(sums, cnt, wc, bc, wf, bf, wo, bo)


def kernel(x, edge_index, edge_attr, batch, W_emb, b_emb, W_conv, b_conv,
           bn_g, bn_b, ln_g, ln_b, W_c2f, b_c2f, W_fc, b_fc, W_out, b_out):
    L = W_conv.shape[0]
    src2 = edge_index[0].reshape(NW, NCH, K).astype(jnp.int32)
    dst2 = edge_index[1].reshape(NW, NCH, K).astype(jnp.int32)
    dst5 = edge_index[1].reshape(NEG, SCH, CK).astype(jnp.int32)
    batch2 = batch.reshape(N, 1).astype(jnp.int32)

    h = _embed(x, W_emb, b_emb.reshape(1, H))
    for l in range(L):
        wd = W_conv[l, :, :H]
        ws = W_conv[l, :, H:2 * H]
        we = W_conv[l, :, 2 * H:]
        p, q = _pq(h, wd, ws)
        ep = _ep(edge_attr, we, b_conv[l].reshape(1, HH))
        z, stats = _edge_z(p, q, ep, dst2, src2)
        m = _gate(z, stats, bn_g[l].reshape(1, HH), bn_b[l].reshape(1, HH))
        (aggp,) = _scatter(m, dst5)
        h = _node_update(aggp, h, ln_g[l].reshape(1, H), ln_b[l].reshape(1, H))

    sums, cnt = _pool(h, batch2)
    out = _head(sums, cnt, W_c2f, b_c2f.reshape(1, -1), W_fc, b_fc,
                W_out, b_out.reshape(1, 1))
    return out
